# baseline (speedup 1.0000x reference)
import jax
import jax.numpy as jnp
from jax.experimental import pallas as pl

_K = 19264
_BK = 512
_NSTEPS = 38


def _fused_kernel(xt_ref, emb_ref, o_ref):
    j = pl.program_id(0)

    @pl.when(j == 0)
    def _init():
        o_ref[...] = jnp.zeros_like(o_ref)

    valid = _K - j * _BK

    xt = xt_ref[...]
    row = jax.lax.broadcasted_iota(jnp.int32, xt.shape, 0)
    xt = jnp.where(row < valid, xt, 0.0)
    y = jnp.log(xt + 1.0).astype(jnp.bfloat16)

    e = emb_ref[...]
    erow = jax.lax.broadcasted_iota(jnp.int32, e.shape, 0)
    e = jnp.where(erow < valid, e, 0.0).astype(jnp.bfloat16)

    o_ref[...] += jax.lax.dot_general(
        y, e, (((0,), (0,)), ((), ())), preferred_element_type=jnp.float32
    )


def kernel(x_seq, bb_gene_emb):
    m, k = x_seq.shape
    _, n = bb_gene_emb.shape
    xt = x_seq.T
    return pl.pallas_call(
        _fused_kernel,
        grid=(_NSTEPS,),
        in_specs=[
            pl.BlockSpec((_BK, m), lambda j: (j, 0)),
            pl.BlockSpec((_BK, n), lambda j: (j, 0)),
        ],
        out_specs=pl.BlockSpec((m, n), lambda j: (0, 0)),
        out_shape=jax.ShapeDtypeStruct((m, n), jnp.float32),
    )(xt, bb_gene_emb)


# speedup vs baseline: 1.0117x; 1.0117x over previous
import jax
import jax.numpy as jnp
from jax.experimental import pallas as pl

_K = 19264
_BM = 2048
_BK = 1024
_NSTEPS = 19


def _fused_kernel(xt_ref, emb_ref, o_ref):
    j = pl.program_id(1)

    @pl.when(j == 0)
    def _init():
        o_ref[...] = jnp.zeros_like(o_ref)

    valid = _K - j * _BK

    xt = xt_ref[...]
    row = jax.lax.broadcasted_iota(jnp.int32, xt.shape, 0)
    xt = jnp.where(row < valid, xt, 0.0)
    y = jnp.log(xt + 1.0).astype(jnp.bfloat16)

    e = emb_ref[...]
    erow = jax.lax.broadcasted_iota(jnp.int32, e.shape, 0)
    e = jnp.where(erow < valid, e, 0.0).astype(jnp.bfloat16)

    o_ref[...] += jax.lax.dot_general(
        y, e, (((0,), (0,)), ((), ())), preferred_element_type=jnp.float32
    )


def kernel(x_seq, bb_gene_emb):
    m, k = x_seq.shape
    _, n = bb_gene_emb.shape
    xt = x_seq.T
    return pl.pallas_call(
        _fused_kernel,
        grid=(m // _BM, _NSTEPS),
        in_specs=[
            pl.BlockSpec((_BK, _BM), lambda i, j: (j, i)),
            pl.BlockSpec((_BK, n), lambda i, j: (j, 0)),
        ],
        out_specs=pl.BlockSpec((_BM, n), lambda i, j: (i, 0)),
        out_shape=jax.ShapeDtypeStruct((m, n), jnp.float32),
    )(xt, bb_gene_emb)
